# baseline (speedup 1.0000x reference)
import jax
import jax.numpy as jnp
from jax import lax
from jax.experimental import pallas as pl
from jax.experimental.pallas import tpu as pltpu
from jax.experimental.pallas import tpu_sc as plsc

N_NODES = 10000
D_FEAT = 128
K_HOPS = 10
HALF = D_FEAT // 2
N_TEC = 16
NP = 10240
RPT = NP // N_TEC
CH = 128
NCH = RPT // CH
EC = 448
N_EDGES = 320000
EPT = N_EDGES // N_TEC
NEC = (-(-EPT // EC) + 1) // 2 * 2
EPT_PAD = NEC * EC
LANES = 16


def _sc_body(x_hbm, epk, coef, out_hbm, g_hbm,
             u_s, deg_s, dq_s,
             pkc, sidx, didx, pkc2, sidx2, didx2, onesb, rbuf, rbuf2,
             gb, hb, zb, dql, db, cb, sem, sem2, sem3, sem4):
    c = lax.axis_index("c")
    t = lax.axis_index("s")
    rowbase = t * RPT
    c32 = (c * NP).astype(jnp.int32)

    zeros16 = jnp.zeros((LANES,), jnp.float32)
    ones16 = jnp.full((LANES,), 1.0, jnp.float32)

    pltpu.sync_copy(coef, cb)

    @pl.loop(0, CH)
    def _zero_zb(r):
        for q in range(HALF // LANES):
            zb[r, pl.ds(q * LANES, LANES)] = zeros16

    @pl.loop(0, EC // LANES)
    def _ones(r):
        onesb[pl.ds(r * LANES, LANES)] = ones16

    @pl.loop(0, RPT // LANES)
    def _zero_dql(r):
        dql[pl.ds(r * LANES, LANES)] = zeros16

    pltpu.sync_copy(dql, deg_s.at[pl.ds(rowbase, RPT)])
    for ch in range(NCH):
        rs = rowbase + ch * CH
        pltpu.sync_copy(zb, u_s.at[pl.ds(rs, CH)])
        pltpu.sync_copy(zb, out_hbm.at[c, pl.ds(rs, CH)])
    plsc.subcore_barrier()

    @pl.loop(0, NEC)
    def _deg(j):
        pltpu.sync_copy(epk.at[t * NEC + j], pkc)
        for q in range(EC // LANES):
            sl = pl.ds(q * LANES, LANES)
            didx[sl] = lax.shift_right_logical(pkc[sl], 16)
        pltpu.sync_copy(onesb, deg_s.at[didx], add=True)

    plsc.subcore_barrier()

    pltpu.sync_copy(deg_s.at[pl.ds(rowbase, RPT)], dql)

    @pl.loop(0, RPT // LANES)
    def _rsq(r):
        sl = pl.ds(r * LANES, LANES)
        d = dql[sl] + 1.0
        y = jnp.full((LANES,), 1.0, jnp.float32)
        for _ in range(7):
            y = jnp.where(y * y * d > 1.0, y * 0.5, y)
        for _ in range(7):
            y = y * (1.5 - 0.5 * d * y * y)
        dql[sl] = y

    pltpu.sync_copy(dql, dq_s.at[pl.ds(rowbase, RPT)])
    plsc.subcore_barrier()

    for ch in range(NCH):
        rs = rowbase + ch * CH
        pltpu.sync_copy(x_hbm.at[c, pl.ds(rs, CH)], gb)
        pltpu.sync_copy(dq_s.at[pl.ds(rs, CH)], db)

        @pl.loop(0, CH // LANES)
        def _g0(g):
            dv16 = db[pl.ds(g * LANES, LANES)]
            for i in range(LANES):
                r = g * LANES + i
                s = dv16[i]
                for q in range(HALF // LANES):
                    sl = pl.ds(q * LANES, LANES)
                    gb[r, sl] = gb[r, sl] * s

        pltpu.sync_copy(gb, g_hbm.at[pl.ds(c * NP + rs, CH)])
    plsc.subcore_barrier()

    @pl.loop(0, K_HOPS)
    def _hop(k):
        ck = cb[k + 1, pl.ds(0, LANES)]

        pltpu.sync_copy(epk.at[t * NEC], pkc)
        for q in range(EC // LANES):
            sl = pl.ds(q * LANES, LANES)
            w = pkc[sl]
            sidx[sl] = (w & jnp.int32(0xFFFF)) + c32
            didx[sl] = lax.shift_right_logical(w, 16)
        pltpu.async_copy(g_hbm.at[sidx], rbuf, sem)

        @pl.loop(0, NEC // 2)
        def _edges(jj):
            j0 = jj * 2
            pltpu.sync_copy(epk.at[t * NEC + j0 + 1], pkc2)
            for q in range(EC // LANES):
                sl = pl.ds(q * LANES, LANES)
                w = pkc2[sl]
                sidx2[sl] = (w & jnp.int32(0xFFFF)) + c32
                didx2[sl] = lax.shift_right_logical(w, 16)
            pltpu.async_copy(g_hbm.at[sidx2], rbuf2, sem2)
            pltpu.make_async_copy(g_hbm.at[sidx], rbuf, sem).wait()
            pltpu.sync_copy(rbuf, u_s.at[didx], add=True)

            @pl.when(jj < NEC // 2 - 1)
            def _():
                pltpu.sync_copy(epk.at[t * NEC + j0 + 2], pkc)
                for q in range(EC // LANES):
                    sl = pl.ds(q * LANES, LANES)
                    w = pkc[sl]
                    sidx[sl] = (w & jnp.int32(0xFFFF)) + c32
                    didx[sl] = lax.shift_right_logical(w, 16)
                pltpu.async_copy(g_hbm.at[sidx], rbuf, sem)

            pltpu.make_async_copy(g_hbm.at[sidx2], rbuf2, sem2).wait()
            pltpu.sync_copy(rbuf2, u_s.at[didx2], add=True)

        plsc.subcore_barrier()

        @pl.loop(0, NCH)
        def _chunk(ch):
            rs = rowbase + ch * CH
            cp1 = pltpu.async_copy(u_s.at[pl.ds(rs, CH)], rbuf.at[pl.ds(0, CH)], sem)
            cp2 = pltpu.async_copy(g_hbm.at[pl.ds(c * NP + rs, CH)], gb, sem2)
            cp3 = pltpu.async_copy(out_hbm.at[c, pl.ds(rs, CH)], hb, sem3)
            cp4 = pltpu.async_copy(dq_s.at[pl.ds(rs, CH)], db, sem4)
            cp1.wait()
            cp2.wait()
            cp3.wait()
            cp4.wait()

            @pl.loop(0, CH // LANES)
            def _ep(g):
                dv16 = db[pl.ds(g * LANES, LANES)]
                for i in range(LANES):
                    r = g * LANES + i
                    s = dv16[i]
                    dv = s * s
                    for q in range(HALF // LANES):
                        sl = pl.ds(q * LANES, LANES)
                        v = rbuf[r, sl] + gb[r, sl]
                        hb[r, sl] = hb[r, sl] + ck * v
                        gb[r, sl] = dv * v

            cp5 = pltpu.async_copy(gb, g_hbm.at[pl.ds(c * NP + rs, CH)], sem2)
            cp6 = pltpu.async_copy(hb, out_hbm.at[c, pl.ds(rs, CH)], sem3)
            cp7 = pltpu.async_copy(zb, u_s.at[pl.ds(rs, CH)], sem)
            cp5.wait()
            cp6.wait()
            cp7.wait()
        plsc.subcore_barrier()

    c0 = cb[0, pl.ds(0, LANES)]

    @pl.loop(0, NCH)
    def _outc(ch):
        rs = rowbase + ch * CH
        pltpu.sync_copy(x_hbm.at[c, pl.ds(rs, CH)], rbuf.at[pl.ds(0, CH)])
        pltpu.sync_copy(out_hbm.at[c, pl.ds(rs, CH)], hb)
        pltpu.sync_copy(dq_s.at[pl.ds(rs, CH)], db)

        @pl.loop(0, CH // LANES)
        def _out(g):
            dv16 = db[pl.ds(g * LANES, LANES)]
            for i in range(LANES):
                r = g * LANES + i
                s = dv16[i]
                for q in range(HALF // LANES):
                    sl = pl.ds(q * LANES, LANES)
                    hb[r, sl] = c0 * rbuf[r, sl] + s * hb[r, sl]

        pltpu.sync_copy(hb, out_hbm.at[c, pl.ds(rs, CH)])


@jax.jit
def _garnoldi_sc(x_pad, epk, coef):
    mesh = plsc.VectorSubcoreMesh(core_axis_name="c", subcore_axis_name="s")
    return pl.kernel(
        _sc_body,
        out_type=(jax.ShapeDtypeStruct((2, NP, HALF), jnp.float32),
                  jax.ShapeDtypeStruct((2 * NP, HALF), jnp.float32)),
        mesh=mesh,
        compiler_params=pltpu.CompilerParams(
            use_tc_tiling_on_sc=False, needs_layout_passes=False),
        scratch_types=[
            pltpu.VMEM_SHARED((NP, HALF), jnp.float32),
            pltpu.VMEM_SHARED((NP,), jnp.float32),
            pltpu.VMEM_SHARED((NP,), jnp.float32),
            pltpu.VMEM((EC,), jnp.int32),
            pltpu.VMEM((EC,), jnp.int32),
            pltpu.VMEM((EC,), jnp.int32),
            pltpu.VMEM((EC,), jnp.int32),
            pltpu.VMEM((EC,), jnp.int32),
            pltpu.VMEM((EC,), jnp.int32),
            pltpu.VMEM((EC,), jnp.float32),
            pltpu.VMEM((EC, HALF), jnp.float32),
            pltpu.VMEM((EC, HALF), jnp.float32),
            pltpu.VMEM((CH, HALF), jnp.float32),
            pltpu.VMEM((CH, HALF), jnp.float32),
            pltpu.VMEM((CH, HALF), jnp.float32),
            pltpu.VMEM((RPT,), jnp.float32),
            pltpu.VMEM((CH,), jnp.float32),
            pltpu.VMEM((16, LANES), jnp.float32),
            pltpu.SemaphoreType.DMA,
            pltpu.SemaphoreType.DMA,
            pltpu.SemaphoreType.DMA,
            pltpu.SemaphoreType.DMA,
        ],
    )(x_pad, epk, coef)


def kernel(x, edge_index, coeffs):
    src = edge_index[0].astype(jnp.int32)
    dst = edge_index[1].astype(jnp.int32)
    pad = N_TEC * EPT_PAD - N_EDGES
    fill = jnp.full((pad,), N_NODES, jnp.int32)
    packed = jnp.concatenate([src, fill]) | (
        jnp.concatenate([dst, fill]) << 16)
    epk = packed.reshape(N_TEC * NEC, EC)

    xh = x.reshape(N_NODES, 2, HALF).transpose(1, 0, 2)
    x_pad = jnp.zeros((2, NP, HALF), jnp.float32).at[:, :N_NODES, :].set(xh)
    coef = jnp.zeros((16,), jnp.float32).at[: K_HOPS + 1].set(coeffs)
    coef = jnp.tile(coef[:, None], (1, LANES))

    out, _ = _garnoldi_sc(x_pad, epk, coef)
    return out[:, :N_NODES, :].transpose(1, 0, 2).reshape(N_NODES, D_FEAT)


# speedup vs baseline: 1.7663x; 1.7663x over previous
import jax
import jax.numpy as jnp
from jax import lax
from jax.experimental import pallas as pl
from jax.experimental.pallas import tpu as pltpu
from jax.experimental.pallas import tpu_sc as plsc

N_NODES = 10000
D_FEAT = 128
K_HOPS = 10
HALF = D_FEAT // 2
N_TEC = 16
NP = 10240
RPT = NP // N_TEC
CH = 128
NCH = RPT // CH
EC = 224
N_EDGES = 320000
EPT = N_EDGES // N_TEC
NEC = (-(-EPT // EC) + 1) // 2 * 2
EPT_PAD = NEC * EC
LANES = 16


def _sc_body(x_hbm, epk, coef, out_hbm, g_hbm,
             u_s, deg_s, dq_s,
             pkc, sidx, didx, pkc2, sidx2, didx2, onesb, rbuf, rbuf2,
             gb, hb, zb, dql, db, cb, sem, sem2, sem3, sem4):
    c = lax.axis_index("c")
    t = lax.axis_index("s")
    rowbase = t * RPT
    c32 = (c * NP).astype(jnp.int32)

    zeros16 = jnp.zeros((LANES,), jnp.float32)
    ones16 = jnp.full((LANES,), 1.0, jnp.float32)

    pltpu.sync_copy(coef, cb)

    @pl.loop(0, CH)
    def _zero_zb(r):
        for q in range(HALF // LANES):
            zb[r, pl.ds(q * LANES, LANES)] = zeros16

    @pl.loop(0, EC // LANES)
    def _ones(r):
        onesb[pl.ds(r * LANES, LANES)] = ones16

    @pl.loop(0, RPT // LANES)
    def _zero_dql(r):
        dql[pl.ds(r * LANES, LANES)] = zeros16

    pltpu.sync_copy(dql, deg_s.at[pl.ds(rowbase, RPT)])
    for ch in range(NCH):
        rs = rowbase + ch * CH
        pltpu.sync_copy(zb, u_s.at[pl.ds(rs, CH)])
        pltpu.sync_copy(zb, out_hbm.at[c, pl.ds(rs, CH)])
    plsc.subcore_barrier()

    @pl.loop(0, NEC)
    def _deg(j):
        pltpu.sync_copy(epk.at[t * NEC + j], pkc)
        for q in range(EC // LANES):
            sl = pl.ds(q * LANES, LANES)
            didx[sl] = lax.shift_right_logical(pkc[sl], 16)
        pltpu.sync_copy(onesb, deg_s.at[didx], add=True)

    plsc.subcore_barrier()

    pltpu.sync_copy(deg_s.at[pl.ds(rowbase, RPT)], dql)

    @pl.loop(0, RPT // LANES)
    def _rsq(r):
        sl = pl.ds(r * LANES, LANES)
        d = dql[sl] + 1.0
        y = jnp.full((LANES,), 1.0, jnp.float32)
        for _ in range(7):
            y = jnp.where(y * y * d > 1.0, y * 0.5, y)
        for _ in range(7):
            y = y * (1.5 - 0.5 * d * y * y)
        dql[sl] = y

    pltpu.sync_copy(dql, dq_s.at[pl.ds(rowbase, RPT)])
    plsc.subcore_barrier()

    for ch in range(NCH):
        rs = rowbase + ch * CH
        pltpu.sync_copy(x_hbm.at[c, pl.ds(rs, CH)], gb)
        pltpu.sync_copy(dq_s.at[pl.ds(rs, CH)], db)

        @pl.loop(0, CH // LANES)
        def _g0(g):
            dv16 = db[pl.ds(g * LANES, LANES)]
            for i in range(LANES):
                r = g * LANES + i
                s = dv16[i]
                for q in range(HALF // LANES):
                    sl = pl.ds(q * LANES, LANES)
                    gb[r, sl] = gb[r, sl] * s

        pltpu.sync_copy(gb, g_hbm.at[pl.ds(c * NP + rs, CH)])
    plsc.subcore_barrier()

    @pl.loop(0, K_HOPS)
    def _hop(k):
        ck = cb[k + 1, pl.ds(0, LANES)]

        pltpu.sync_copy(epk.at[t * NEC], pkc)
        for q in range(EC // LANES):
            sl = pl.ds(q * LANES, LANES)
            w = pkc[sl]
            sidx[sl] = (w & jnp.int32(0xFFFF)) + c32
            didx[sl] = lax.shift_right_logical(w, 16)
        pltpu.async_copy(g_hbm.at[sidx], rbuf, sem)

        @pl.loop(0, NEC // 2)
        def _edges(jj):
            j0 = jj * 2
            pltpu.sync_copy(epk.at[t * NEC + j0 + 1], pkc2)
            for q in range(EC // LANES):
                sl = pl.ds(q * LANES, LANES)
                w = pkc2[sl]
                sidx2[sl] = (w & jnp.int32(0xFFFF)) + c32
                didx2[sl] = lax.shift_right_logical(w, 16)
            pltpu.async_copy(g_hbm.at[sidx2], rbuf2, sem2)
            pltpu.make_async_copy(g_hbm.at[sidx], rbuf, sem).wait()
            pltpu.sync_copy(rbuf, u_s.at[didx], add=True)

            @pl.when(jj < NEC // 2 - 1)
            def _():
                pltpu.sync_copy(epk.at[t * NEC + j0 + 2], pkc)
                for q in range(EC // LANES):
                    sl = pl.ds(q * LANES, LANES)
                    w = pkc[sl]
                    sidx[sl] = (w & jnp.int32(0xFFFF)) + c32
                    didx[sl] = lax.shift_right_logical(w, 16)
                pltpu.async_copy(g_hbm.at[sidx], rbuf, sem)

            pltpu.make_async_copy(g_hbm.at[sidx2], rbuf2, sem2).wait()
            pltpu.sync_copy(rbuf2, u_s.at[didx2], add=True)

        plsc.subcore_barrier()

        @pl.loop(0, NCH)
        def _chunk(ch):
            rs = rowbase + ch * CH
            cp1 = pltpu.async_copy(u_s.at[pl.ds(rs, CH)], rbuf.at[pl.ds(0, CH)], sem)
            cp2 = pltpu.async_copy(g_hbm.at[pl.ds(c * NP + rs, CH)], gb, sem2)
            cp3 = pltpu.async_copy(out_hbm.at[c, pl.ds(rs, CH)], hb, sem3)
            cp4 = pltpu.async_copy(dq_s.at[pl.ds(rs, CH)], db, sem4)
            cp1.wait()
            cp2.wait()
            cp3.wait()
            cp4.wait()

            @pl.loop(0, CH // LANES)
            def _ep(g):
                dv16 = db[pl.ds(g * LANES, LANES)]
                for i in range(LANES):
                    r = g * LANES + i
                    s = dv16[i]
                    dv = s * s
                    for q in range(HALF // LANES):
                        sl = pl.ds(q * LANES, LANES)
                        v = rbuf[r, sl] + gb[r, sl]
                        hb[r, sl] = hb[r, sl] + ck * v
                        gb[r, sl] = dv * v

            cp5 = pltpu.async_copy(gb, g_hbm.at[pl.ds(c * NP + rs, CH)], sem2)
            cp6 = pltpu.async_copy(hb, out_hbm.at[c, pl.ds(rs, CH)], sem3)
            cp7 = pltpu.async_copy(zb, u_s.at[pl.ds(rs, CH)], sem)
            cp5.wait()
            cp6.wait()
            cp7.wait()
        plsc.subcore_barrier()

    c0 = cb[0, pl.ds(0, LANES)]

    @pl.loop(0, NCH)
    def _outc(ch):
        rs = rowbase + ch * CH
        pltpu.sync_copy(x_hbm.at[c, pl.ds(rs, CH)], rbuf.at[pl.ds(0, CH)])
        pltpu.sync_copy(out_hbm.at[c, pl.ds(rs, CH)], hb)
        pltpu.sync_copy(dq_s.at[pl.ds(rs, CH)], db)

        @pl.loop(0, CH // LANES)
        def _out(g):
            dv16 = db[pl.ds(g * LANES, LANES)]
            for i in range(LANES):
                r = g * LANES + i
                s = dv16[i]
                for q in range(HALF // LANES):
                    sl = pl.ds(q * LANES, LANES)
                    hb[r, sl] = c0 * rbuf[r, sl] + s * hb[r, sl]

        pltpu.sync_copy(hb, out_hbm.at[c, pl.ds(rs, CH)])


@jax.jit
def _garnoldi_sc(x_pad, epk, coef):
    mesh = plsc.VectorSubcoreMesh(core_axis_name="c", subcore_axis_name="s")
    return pl.kernel(
        _sc_body,
        out_type=(jax.ShapeDtypeStruct((2, NP, HALF), jnp.float32),
                  jax.ShapeDtypeStruct((2 * NP, HALF), jnp.float32)),
        mesh=mesh,
        compiler_params=pltpu.CompilerParams(
            use_tc_tiling_on_sc=False, needs_layout_passes=False),
        scratch_types=[
            pltpu.VMEM_SHARED((NP, HALF), jnp.float32),
            pltpu.VMEM_SHARED((NP,), jnp.float32),
            pltpu.VMEM_SHARED((NP,), jnp.float32),
            pltpu.VMEM((EC,), jnp.int32),
            pltpu.VMEM((EC,), jnp.int32),
            pltpu.VMEM((EC,), jnp.int32),
            pltpu.VMEM((EC,), jnp.int32),
            pltpu.VMEM((EC,), jnp.int32),
            pltpu.VMEM((EC,), jnp.int32),
            pltpu.VMEM((EC,), jnp.float32),
            pltpu.VMEM((EC, HALF), jnp.float32),
            pltpu.VMEM((EC, HALF), jnp.float32),
            pltpu.VMEM((CH, HALF), jnp.float32),
            pltpu.VMEM((CH, HALF), jnp.float32),
            pltpu.VMEM((CH, HALF), jnp.float32),
            pltpu.VMEM((RPT,), jnp.float32),
            pltpu.VMEM((CH,), jnp.float32),
            pltpu.VMEM((16, LANES), jnp.float32),
            pltpu.SemaphoreType.DMA,
            pltpu.SemaphoreType.DMA,
            pltpu.SemaphoreType.DMA,
            pltpu.SemaphoreType.DMA,
        ],
    )(x_pad, epk, coef)


def kernel(x, edge_index, coeffs):
    src = edge_index[0].astype(jnp.int32)
    dst = edge_index[1].astype(jnp.int32)
    pad = N_TEC * EPT_PAD - N_EDGES
    fill = jnp.full((pad,), N_NODES, jnp.int32)
    packed = jnp.concatenate([src, fill]) | (
        jnp.concatenate([dst, fill]) << 16)
    epk = packed.reshape(N_TEC * NEC, EC)

    xh = x.reshape(N_NODES, 2, HALF).transpose(1, 0, 2)
    x_pad = jnp.zeros((2, NP, HALF), jnp.float32).at[:, :N_NODES, :].set(xh)
    coef = jnp.zeros((16,), jnp.float32).at[: K_HOPS + 1].set(coeffs)
    coef = jnp.tile(coef[:, None], (1, LANES))

    out, _ = _garnoldi_sc(x_pad, epk, coef)
    return out[:, :N_NODES, :].transpose(1, 0, 2).reshape(N_NODES, D_FEAT)
